# Initial kernel scaffold; baseline (speedup 1.0000x reference)
import jax
import jax.numpy as jnp
from jax.experimental import pallas as pl


def kernel(input_text, table):
    raise NotImplementedError("write your pallas kernel here")


# speedup vs baseline: 1.6830x; 1.6830x over previous
import functools

import jax
import jax.numpy as jnp
from jax import lax
from jax.experimental import pallas as pl
from jax.experimental.pallas import tpu as pltpu
from jax.experimental.pallas import tpu_sc as plsc

EMBED_DIM = 64
CHUNK = 128


@functools.lru_cache(maxsize=None)
def _make_gather(n_rows, rows_per_worker, d):
    mesh = plsc.VectorSubcoreMesh(core_axis_name="c", subcore_axis_name="s")
    info = plsc.get_sparse_core_info()
    nc = info.num_cores

    @functools.partial(
        pl.kernel,
        mesh=mesh,
        out_type=jax.ShapeDtypeStruct((n_rows, CHUNK, d), jnp.float32),
        scratch_types=[
            pltpu.VMEM((rows_per_worker, CHUNK), jnp.int32),
            pltpu.VMEM((CHUNK, d), jnp.float32),
            pltpu.SemaphoreType.DMA,
        ],
        compiler_params=pltpu.CompilerParams(use_tc_tiling_on_sc=False),
    )
    def k(idx_hbm, table_hbm, out_hbm, idx_v, rows_v, sem):
        wid = lax.axis_index("s") * nc + lax.axis_index("c")
        base = wid * rows_per_worker
        pltpu.sync_copy(idx_hbm.at[pl.ds(base, rows_per_worker)], idx_v)

        def body(j, carry):
            pltpu.async_copy(table_hbm.at[idx_v.at[j]], rows_v, sem).wait()
            pltpu.sync_copy(rows_v, out_hbm.at[base + j])
            return carry

        lax.fori_loop(0, rows_per_worker, body, 0)

    return k


def kernel(input_text, table):
    b, h = input_text.shape
    _, d = table.shape
    total = b * h
    n_rows = total // CHUNK
    rows_per_worker = n_rows // 32
    idx = input_text.reshape(n_rows, CHUNK).astype(jnp.int32)
    out = _make_gather(n_rows, rows_per_worker, d)(idx, table)
    return out.reshape(b, h, d)


# speedup vs baseline: 1.8778x; 1.1157x over previous
import functools

import jax
import jax.numpy as jnp
from jax import lax
from jax.experimental import pallas as pl
from jax.experimental.pallas import tpu as pltpu
from jax.experimental.pallas import tpu_sc as plsc

EMBED_DIM = 64
CHUNK = 128


@functools.lru_cache(maxsize=None)
def _make_gather(n_rows, rows_per_worker, d):
    mesh = plsc.VectorSubcoreMesh(core_axis_name="c", subcore_axis_name="s")
    info = plsc.get_sparse_core_info()
    nc = info.num_cores

    M = 8
    K = 4
    assert rows_per_worker % M == 0 and rows_per_worker > M + K

    @functools.partial(
        pl.kernel,
        mesh=mesh,
        out_type=jax.ShapeDtypeStruct((n_rows, CHUNK, d), jnp.float32),
        scratch_types=[
            pltpu.VMEM((rows_per_worker, CHUNK), jnp.int32),
            pltpu.VMEM((M, CHUNK, d), jnp.float32),
            pltpu.SemaphoreType.DMA((M,)),
            pltpu.SemaphoreType.DMA((M,)),
        ],
        compiler_params=pltpu.CompilerParams(use_tc_tiling_on_sc=False),
    )
    def k(idx_hbm, table_hbm, out_hbm, idx_v, rows_v, gsem, wsem):
        wid = lax.axis_index("s") * nc + lax.axis_index("c")
        base = wid * rows_per_worker
        pltpu.sync_copy(idx_hbm.at[pl.ds(base, rows_per_worker)], idx_v)

        def gather(j, b):
            pltpu.async_copy(table_hbm.at[idx_v.at[j]], rows_v.at[b],
                             gsem.at[b])

        def write(j, b):
            pltpu.make_async_copy(rows_v.at[b], out_hbm.at[base + j],
                                  gsem.at[b]).wait()
            pltpu.async_copy(rows_v.at[b], out_hbm.at[base + j], wsem.at[b])

        for j in range(M):
            gather(j, j % M)
            if j >= K:
                write(j - K, (j - K) % M)

        def group(g, carry):
            for b in range(M):
                j = g * M + b
                pltpu.make_async_copy(rows_v.at[b], out_hbm.at[0],
                                      wsem.at[b]).wait()
                gather(j, b)
                b2 = (b - K) % M
                write(j - K, b2)
            return carry

        lax.fori_loop(1, rows_per_worker // M, group, 0)

        for jj in range(rows_per_worker - K, rows_per_worker):
            write(jj, jj % M)
        for b in range(M):
            pltpu.make_async_copy(rows_v.at[b], out_hbm.at[0],
                                  wsem.at[b]).wait()

    return k


def kernel(input_text, table):
    b, h = input_text.shape
    _, d = table.shape
    total = b * h
    n_rows = total // CHUNK
    rows_per_worker = n_rows // 32
    idx = input_text.reshape(n_rows, CHUNK).astype(jnp.int32)
    out = _make_gather(n_rows, rows_per_worker, d)(idx, table)
    return out.reshape(b, h, d)
